# baseline (speedup 1.0000x reference)
import functools

import jax
import jax.numpy as jnp
from jax import lax
from jax.experimental import pallas as pl
from jax.experimental.pallas import tpu as pltpu
from jax.experimental.pallas import tpu_sc as plsc

_N = 5000
_NP = 5120
_NCHUNKS = _NP // 128
_CONF_T = 0.7
_NMS_T = 0.65


def _score_body(zone_ref, pred_ref, out_ref):
    p = pred_ref[...]
    cx = p[:, 0:1]
    cy = p[:, 1:2]
    w = p[:, 2:3]
    h = p[:, 3:4]
    obj = p[:, 4:5]
    x1 = cx - w / 2
    y1 = cy - h / 2
    x2 = cx + w / 2
    y2 = cy + h / 2

    lane = lax.broadcasted_iota(jnp.int32, (_NP, 128), 1)
    clsmask = (lane >= 5) & (lane < 85)
    masked = jnp.where(clsmask, p, -jnp.inf)
    cc = jnp.max(masked, axis=1, keepdims=True)
    eqm = clsmask & (p == cc)
    cls_idx = jnp.min(jnp.where(eqm, lane, 1 << 20), axis=1, keepdims=True) - 5
    cp = cls_idx.astype(jnp.float32)

    conf = obj * cc
    valid = (conf >= _CONF_T).astype(jnp.float32)
    sortkey = jnp.where(valid > 0, conf, jnp.float32(-1e30))
    score = jnp.maximum(obj, cc)
    ctrx = (x1 + x2) / 2
    ctry = (y1 + y2) / 2

    cnt = jnp.zeros((_NP, 1), jnp.float32)
    for j in range(8):
        xi = zone_ref[j, 0]
        yi = zone_ref[j, 1]
        xj = zone_ref[(j - 1) % 8, 0]
        yj = zone_ref[(j - 1) % 8, 1]
        gyi = yi > ctry
        gyj = yj > ctry
        gx = (xj - xi) * (ctry - yi) / (yj - yi) + xi
        m = (gyi != gyj) & (gx > ctrx)
        cnt = cnt + jnp.where(m, 1.0, 0.0)
    inz = ((cnt.astype(jnp.int32) & 1) > 0).astype(jnp.float32)

    pad = jnp.zeros((_NP, 128 - 11), jnp.float32)
    out_ref[...] = jnp.concatenate(
        [y1, x1, y2, x2, inz, score, cp, ctry, ctrx, valid, sortkey, pad],
        axis=1,
    )


def _nms_body(bcol_ref, brow_ref, vrow_ref, keep_ref):
    vrow = vrow_ref[...]
    keep_ref[...] = vrow
    nvalid = jnp.sum(vrow).astype(jnp.int32)

    x1r = brow_ref[0:1, :]
    y1r = brow_ref[1:2, :]
    x2r = brow_ref[2:3, :]
    y2r = brow_ref[3:4, :]
    arear = jnp.maximum(x2r - x1r, 0.0) * jnp.maximum(y2r - y1r, 0.0)
    lane = lax.broadcasted_iota(jnp.int32, (128, _NP), 1)
    subl = lax.broadcasted_iota(jnp.int32, (128, _NP), 0)

    for c in range(_NCHUNKS):
        off = c * 128

        @pl.when(off < nvalid)
        def _(off=off):
            x1c = bcol_ref[off:off + 128, 0:1]
            y1c = bcol_ref[off:off + 128, 1:2]
            x2c = bcol_ref[off:off + 128, 2:3]
            y2c = bcol_ref[off:off + 128, 3:4]
            areac = jnp.maximum(x2c - x1c, 0.0) * jnp.maximum(y2c - y1c, 0.0)
            ltx = jnp.maximum(x1c, x1r)
            lty = jnp.maximum(y1c, y1r)
            rbx = jnp.minimum(x2c, x2r)
            rby = jnp.minimum(y2c, y2r)
            inter = jnp.maximum(rbx - ltx, 0.0) * jnp.maximum(rby - lty, 0.0)
            union = areac + arear - inter
            iou = inter / jnp.maximum(union, 1e-9)
            conf = ((iou > _NMS_T) & (lane > subl + off)).astype(jnp.float32)
            conf_cc = conf[:, off:off + 128]

            b = keep_ref[0:1, off:off + 128]
            b8 = jnp.broadcast_to(b, (8, 128))

            def cond(carry):
                return carry[1]

            def body(carry):
                k, _ = carry
                sup = jnp.dot(k, conf_cc, preferred_element_type=jnp.float32)
                kn = b8 * (1.0 - (sup > 0.5).astype(jnp.float32))
                return kn, jnp.any(kn != k)

            k, _ = lax.while_loop(cond, body, (b8, jnp.bool_(True)))

            sup_all = jnp.dot(k, conf, preferred_element_type=jnp.float32)
            keep_ref[...] = keep_ref[...] * (
                1.0 - (sup_all[0:1, :] > 0.5).astype(jnp.float32))


def _score_call(zone, predp):
    return pl.pallas_call(
        _score_body,
        out_shape=jax.ShapeDtypeStruct((_NP, 128), jnp.float32),
        in_specs=[
            pl.BlockSpec(memory_space=pltpu.SMEM),
            pl.BlockSpec(memory_space=pltpu.VMEM),
        ],
        out_specs=pl.BlockSpec(memory_space=pltpu.VMEM),
    )(zone, predp)


_NW = 32
_ROWS_W = _NP // _NW
_IDX_CH = 2
_IDX_W = _ROWS_W // _IDX_CH


def _gather_call(table, idx2d):
    mesh = plsc.VectorSubcoreMesh(core_axis_name="c", subcore_axis_name="s")

    @functools.partial(
        pl.kernel,
        mesh=mesh,
        out_type=jax.ShapeDtypeStruct((_NP, 128), jnp.float32),
        scratch_types=[
            pltpu.VMEM((_IDX_CH, _IDX_W), jnp.int32),
            pltpu.VMEM((_ROWS_W, 128), jnp.float32),
            pltpu.SemaphoreType.DMA,
        ],
    )
    def k(table_hbm, idx_hbm, out_hbm, idx_v, rows_v, sem):
        wid = lax.axis_index("s") * 2 + lax.axis_index("c")
        base = wid * _ROWS_W
        pltpu.sync_copy(idx_hbm.at[pl.ds(wid * _IDX_CH, _IDX_CH)], idx_v)
        for j in range(_IDX_CH):
            pltpu.async_copy(
                table_hbm.at[idx_v.at[j]],
                rows_v.at[pl.ds(j * _IDX_W, _IDX_W)],
                sem,
            ).wait()
        pltpu.sync_copy(rows_v, out_hbm.at[pl.ds(base, _ROWS_W)])

    return k(table, idx2d)


def _nms_call(bcol, brow, vrow):
    return pl.pallas_call(
        _nms_body,
        out_shape=jax.ShapeDtypeStruct((1, _NP), jnp.float32),
        in_specs=[
            pl.BlockSpec(memory_space=pltpu.VMEM),
            pl.BlockSpec(memory_space=pltpu.VMEM),
            pl.BlockSpec(memory_space=pltpu.VMEM),
        ],
        out_specs=pl.BlockSpec(memory_space=pltpu.VMEM),
    )(bcol, brow, vrow)


def kernel(prediction, zone):
    pred = prediction[0]
    predp = jnp.pad(pred, ((0, _NP - _N), (0, 128 - 85)))
    feats = _score_call(zone, predp)

    order = jnp.argsort(-feats[:_N, 10], stable=True).astype(jnp.int32)
    order_p = jnp.concatenate(
        [order, jnp.full((_NP - _N,), _N, jnp.int32)])
    s = _gather_call(feats, order_p.reshape(_NW * _IDX_CH, _IDX_W))

    bcol = jnp.stack([s[:, 1], s[:, 0], s[:, 3], s[:, 2]], axis=1)
    brow = bcol.T
    vrow = s[:, 9].reshape(1, _NP)
    keep = _nms_call(bcol, brow, vrow)

    boxes_yxyx = s[:_N, 0:4]
    in_zone = s[:_N, 4] > 0.5
    scores = s[:_N, 5]
    classes = s[:_N, 6].astype(jnp.int32)
    centers_yx = s[:_N, 7:9]
    keep_b = keep[0, :_N] > 0.5
    return (boxes_yxyx, in_zone, scores, classes, centers_yx, keep_b)


# speedup vs baseline: 1.2584x; 1.2584x over previous
import functools

import jax
import jax.numpy as jnp
from jax import lax
from jax.experimental import pallas as pl
from jax.experimental.pallas import tpu as pltpu
from jax.experimental.pallas import tpu_sc as plsc

_N = 5000
_NP = 5120
_NCHUNKS = _NP // 128
_CONF_T = 0.7
_NMS_T = 0.65


def _score_body(zone_ref, predt_ref, out_ref):
    cx = predt_ref[0:1, :]
    cy = predt_ref[1:2, :]
    w = predt_ref[2:3, :]
    h = predt_ref[3:4, :]
    obj = predt_ref[4:5, :]
    x1 = cx - w / 2
    y1 = cy - h / 2
    x2 = cx + w / 2
    y2 = cy + h / 2

    p = predt_ref[...]
    row = lax.broadcasted_iota(jnp.int32, (128, _NP), 0)
    clsmask = (row >= 5) & (row < 85)
    masked = jnp.where(clsmask, p, -jnp.inf)
    cc = jnp.max(masked, axis=0, keepdims=True)
    eqm = clsmask & (p == cc)
    cls_idx = jnp.min(jnp.where(eqm, row, 1 << 20), axis=0, keepdims=True) - 5
    cp = cls_idx.astype(jnp.float32)

    conf = obj * cc
    valid = (conf >= _CONF_T).astype(jnp.float32)
    sortkey = jnp.where(valid > 0, conf, jnp.float32(-1e30))
    score = jnp.maximum(obj, cc)
    ctrx = (x1 + x2) / 2
    ctry = (y1 + y2) / 2

    cnt = jnp.zeros((1, _NP), jnp.float32)
    for j in range(8):
        xi = zone_ref[j, 0]
        yi = zone_ref[j, 1]
        xj = zone_ref[(j - 1) % 8, 0]
        yj = zone_ref[(j - 1) % 8, 1]
        gyi = yi > ctry
        gyj = yj > ctry
        gx = (xj - xi) * (ctry - yi) / (yj - yi) + xi
        m = (gyi != gyj) & (gx > ctrx)
        cnt = cnt + jnp.where(m, 1.0, 0.0)
    inz = ((cnt.astype(jnp.int32) & 1) > 0).astype(jnp.float32)

    pad = jnp.zeros((5, _NP), jnp.float32)
    out_ref[...] = jnp.concatenate(
        [y1, x1, y2, x2, inz, score, cp, ctry, ctrx, valid, sortkey, pad],
        axis=0,
    )


def _nms_chunks(bcol_ref, keep_ref, boxr, nvalid, nchunks, width):
    x1r, y1r, x2r, y2r = boxr
    arear = jnp.maximum(x2r - x1r, 0.0) * jnp.maximum(y2r - y1r, 0.0)
    lane = lax.broadcasted_iota(jnp.int32, (128, width), 1)
    subl = lax.broadcasted_iota(jnp.int32, (128, width), 0)

    for c in range(nchunks):
        off = c * 128

        @pl.when(off < nvalid)
        def _(off=off):
            x1c = bcol_ref[off:off + 128, 0:1]
            y1c = bcol_ref[off:off + 128, 1:2]
            x2c = bcol_ref[off:off + 128, 2:3]
            y2c = bcol_ref[off:off + 128, 3:4]
            areac = jnp.maximum(x2c - x1c, 0.0) * jnp.maximum(y2c - y1c, 0.0)
            ltx = jnp.maximum(x1c, x1r)
            lty = jnp.maximum(y1c, y1r)
            rbx = jnp.minimum(x2c, x2r)
            rby = jnp.minimum(y2c, y2r)
            inter = jnp.maximum(rbx - ltx, 0.0) * jnp.maximum(rby - lty, 0.0)
            union = areac + arear - inter
            iou = inter / jnp.maximum(union, 1e-9)
            conf = ((iou > _NMS_T) & (lane > subl + off)).astype(jnp.float32)
            conf_cc = conf[:, off:off + 128]

            b = keep_ref[0:1, off:off + 128]
            b8 = jnp.broadcast_to(b, (8, 128))

            def cond(carry):
                return carry[1]

            def body(carry):
                k, _ = carry
                sup = jnp.dot(k, conf_cc, preferred_element_type=jnp.float32)
                kn = b8 * (1.0 - (sup > 0.5).astype(jnp.float32))
                return kn, jnp.any(kn != k)

            k, _ = lax.while_loop(cond, body, (b8, jnp.bool_(True)))

            sup_all = jnp.dot(k, conf, preferred_element_type=jnp.float32)
            keep_ref[0:1, 0:width] = keep_ref[0:1, 0:width] * (
                1.0 - (sup_all[0:1, :] > 0.5).astype(jnp.float32))


_TIER = 2048


def _nms_body(bcol_ref, brow_ref, vrow_ref, keep_ref):
    vrow = vrow_ref[...]
    keep_ref[...] = vrow
    nvalid = jnp.sum(vrow).astype(jnp.int32)

    x1r = brow_ref[0:1, :]
    y1r = brow_ref[1:2, :]
    x2r = brow_ref[2:3, :]
    y2r = brow_ref[3:4, :]

    @pl.when(nvalid <= _TIER)
    def _():
        boxr = (x1r[:, :_TIER], y1r[:, :_TIER], x2r[:, :_TIER],
                y2r[:, :_TIER])
        _nms_chunks(bcol_ref, keep_ref, boxr, nvalid, _TIER // 128, _TIER)

    @pl.when(nvalid > _TIER)
    def _():
        _nms_chunks(bcol_ref, keep_ref, (x1r, y1r, x2r, y2r), nvalid,
                    _NCHUNKS, _NP)


def _score_call(zone, predt):
    return pl.pallas_call(
        _score_body,
        out_shape=jax.ShapeDtypeStruct((16, _NP), jnp.float32),
        in_specs=[
            pl.BlockSpec(memory_space=pltpu.SMEM),
            pl.BlockSpec(memory_space=pltpu.VMEM),
        ],
        out_specs=pl.BlockSpec(memory_space=pltpu.VMEM),
    )(zone, predt)


_NW = 32
_ROWS_W = _NP // _NW
_IDX_CH = 2
_IDX_W = _ROWS_W // _IDX_CH


def _gather_call(table, idx2d):
    mesh = plsc.VectorSubcoreMesh(core_axis_name="c", subcore_axis_name="s")

    @functools.partial(
        pl.kernel,
        mesh=mesh,
        out_type=jax.ShapeDtypeStruct((_NP, 128), jnp.float32),
        scratch_types=[
            pltpu.VMEM((_IDX_CH, _IDX_W), jnp.int32),
            pltpu.VMEM((_ROWS_W, 128), jnp.float32),
            pltpu.SemaphoreType.DMA,
        ],
    )
    def k(table_hbm, idx_hbm, out_hbm, idx_v, rows_v, sem):
        wid = lax.axis_index("s") * 2 + lax.axis_index("c")
        base = wid * _ROWS_W
        pltpu.sync_copy(idx_hbm.at[pl.ds(wid * _IDX_CH, _IDX_CH)], idx_v)
        for j in range(_IDX_CH):
            pltpu.async_copy(
                table_hbm.at[idx_v.at[j]],
                rows_v.at[pl.ds(j * _IDX_W, _IDX_W)],
                sem,
            ).wait()
        pltpu.sync_copy(rows_v, out_hbm.at[pl.ds(base, _ROWS_W)])

    return k(table, idx2d)


def _nms_call(bcol, brow, vrow):
    return pl.pallas_call(
        _nms_body,
        out_shape=jax.ShapeDtypeStruct((1, _NP), jnp.float32),
        in_specs=[
            pl.BlockSpec(memory_space=pltpu.VMEM),
            pl.BlockSpec(memory_space=pltpu.VMEM),
            pl.BlockSpec(memory_space=pltpu.VMEM),
        ],
        out_specs=pl.BlockSpec(memory_space=pltpu.VMEM),
    )(bcol, brow, vrow)


def kernel(prediction, zone):
    pred = prediction[0]
    predt = jnp.pad(pred, ((0, _NP - _N), (0, 128 - 85))).T
    featst = _score_call(zone, predt)

    order = jnp.argsort(-featst[10, :_N], stable=True).astype(jnp.int32)
    order_p = jnp.concatenate(
        [order, jnp.full((_NP - _N,), _N, jnp.int32)])
    table = jnp.pad(featst.T, ((0, 0), (0, 112)))
    s = _gather_call(table, order_p.reshape(_NW * _IDX_CH, _IDX_W))

    bcol = jnp.stack([s[:, 1], s[:, 0], s[:, 3], s[:, 2]], axis=1)
    brow = bcol.T
    vrow = s[:, 9].reshape(1, _NP)
    keep = _nms_call(bcol, brow, vrow)

    boxes_yxyx = s[:_N, 0:4]
    in_zone = s[:_N, 4] > 0.5
    scores = s[:_N, 5]
    classes = s[:_N, 6].astype(jnp.int32)
    centers_yx = s[:_N, 7:9]
    keep_b = keep[0, :_N] > 0.5
    return (boxes_yxyx, in_zone, scores, classes, centers_yx, keep_b)


# speedup vs baseline: 1.2632x; 1.0038x over previous
import functools

import jax
import jax.numpy as jnp
from jax import lax
from jax.experimental import pallas as pl
from jax.experimental.pallas import tpu as pltpu
from jax.experimental.pallas import tpu_sc as plsc

_N = 5000
_NP = 5120
_NCHUNKS = _NP // 128
_CONF_T = 0.7
_NMS_T = 0.65


def _score_body(zone_ref, predt_ref, out_ref):
    cx = predt_ref[0:1, :]
    cy = predt_ref[1:2, :]
    w = predt_ref[2:3, :]
    h = predt_ref[3:4, :]
    obj = predt_ref[4:5, :]
    x1 = cx - w / 2
    y1 = cy - h / 2
    x2 = cx + w / 2
    y2 = cy + h / 2

    p = predt_ref[...]
    row = lax.broadcasted_iota(jnp.int32, (128, _NP), 0)
    clsmask = (row >= 5) & (row < 85)
    masked = jnp.where(clsmask, p, -jnp.inf)
    cc = jnp.max(masked, axis=0, keepdims=True)
    eqm = clsmask & (p == cc)
    cls_idx = jnp.min(jnp.where(eqm, row, 1 << 20), axis=0, keepdims=True) - 5
    cp = cls_idx.astype(jnp.float32)

    conf = obj * cc
    valid = (conf >= _CONF_T).astype(jnp.float32)
    sortkey = jnp.where(valid > 0, conf, jnp.float32(-1e30))
    score = jnp.maximum(obj, cc)
    ctrx = (x1 + x2) / 2
    ctry = (y1 + y2) / 2

    cnt = jnp.zeros((1, _NP), jnp.float32)
    for j in range(8):
        xi = zone_ref[j, 0]
        yi = zone_ref[j, 1]
        xj = zone_ref[(j - 1) % 8, 0]
        yj = zone_ref[(j - 1) % 8, 1]
        gyi = yi > ctry
        gyj = yj > ctry
        gx = (xj - xi) * (ctry - yi) / (yj - yi) + xi
        m = (gyi != gyj) & (gx > ctrx)
        cnt = cnt + jnp.where(m, 1.0, 0.0)
    inz = ((cnt.astype(jnp.int32) & 1) > 0).astype(jnp.float32)

    pad = jnp.zeros((5, _NP), jnp.float32)
    out_ref[...] = jnp.concatenate(
        [y1, x1, y2, x2, inz, score, cp, ctry, ctrx, valid, sortkey, pad],
        axis=0,
    )


def _nms_chunks(bcol_ref, keep_ref, boxr, nvalid, nchunks, width):
    x1r, y1r, x2r, y2r = boxr
    arear_full = jnp.maximum(x2r - x1r, 0.0) * jnp.maximum(y2r - y1r, 0.0)

    for c in range(nchunks):
        off = c * 128
        sw = width - off

        @pl.when(off < nvalid)
        def _(off=off, sw=sw):
            x1c = bcol_ref[off:off + 128, 0:1]
            y1c = bcol_ref[off:off + 128, 1:2]
            x2c = bcol_ref[off:off + 128, 2:3]
            y2c = bcol_ref[off:off + 128, 3:4]
            areac = jnp.maximum(x2c - x1c, 0.0) * jnp.maximum(y2c - y1c, 0.0)
            ltx = jnp.maximum(x1c, x1r[:, off:width])
            lty = jnp.maximum(y1c, y1r[:, off:width])
            rbx = jnp.minimum(x2c, x2r[:, off:width])
            rby = jnp.minimum(y2c, y2r[:, off:width])
            inter = jnp.maximum(rbx - ltx, 0.0) * jnp.maximum(rby - lty, 0.0)
            union = areac + arear_full[:, off:width] - inter
            iou = inter / jnp.maximum(union, 1e-9)
            lane = lax.broadcasted_iota(jnp.int32, (128, sw), 1)
            subl = lax.broadcasted_iota(jnp.int32, (128, sw), 0)
            conf = ((iou > _NMS_T) & (lane > subl)).astype(jnp.float32)
            conf_cc = conf[:, 0:128]

            b = keep_ref[0:1, off:off + 128]
            b8 = jnp.broadcast_to(b, (8, 128))

            def cond(carry):
                return carry[1]

            def body(carry):
                k, _ = carry
                sup = jnp.dot(k, conf_cc, preferred_element_type=jnp.float32)
                kn = b8 * (1.0 - (sup > 0.5).astype(jnp.float32))
                return kn, jnp.any(kn != k)

            k, _ = lax.while_loop(cond, body, (b8, jnp.bool_(True)))

            sup_all = jnp.dot(k, conf, preferred_element_type=jnp.float32)
            keep_ref[0:1, off:width] = keep_ref[0:1, off:width] * (
                1.0 - (sup_all[0:1, :] > 0.5).astype(jnp.float32))


_TIER = 2048


def _nms_body(bcol_ref, brow_ref, vrow_ref, keep_ref):
    vrow = vrow_ref[...]
    keep_ref[...] = vrow
    nvalid = jnp.sum(vrow).astype(jnp.int32)

    x1r = brow_ref[0:1, :]
    y1r = brow_ref[1:2, :]
    x2r = brow_ref[2:3, :]
    y2r = brow_ref[3:4, :]

    @pl.when(nvalid <= _TIER)
    def _():
        boxr = (x1r[:, :_TIER], y1r[:, :_TIER], x2r[:, :_TIER],
                y2r[:, :_TIER])
        _nms_chunks(bcol_ref, keep_ref, boxr, nvalid, _TIER // 128, _TIER)

    @pl.when(nvalid > _TIER)
    def _():
        _nms_chunks(bcol_ref, keep_ref, (x1r, y1r, x2r, y2r), nvalid,
                    _NCHUNKS, _NP)


def _score_call(zone, predt):
    return pl.pallas_call(
        _score_body,
        out_shape=jax.ShapeDtypeStruct((16, _NP), jnp.float32),
        in_specs=[
            pl.BlockSpec(memory_space=pltpu.SMEM),
            pl.BlockSpec(memory_space=pltpu.VMEM),
        ],
        out_specs=pl.BlockSpec(memory_space=pltpu.VMEM),
    )(zone, predt)


_NW = 32
_ROWS_W = _NP // _NW
_IDX_CH = 2
_IDX_W = _ROWS_W // _IDX_CH


def _gather_call(table, idx2d):
    mesh = plsc.VectorSubcoreMesh(core_axis_name="c", subcore_axis_name="s")

    @functools.partial(
        pl.kernel,
        mesh=mesh,
        out_type=jax.ShapeDtypeStruct((_NP, 128), jnp.float32),
        scratch_types=[
            pltpu.VMEM((_IDX_CH, _IDX_W), jnp.int32),
            pltpu.VMEM((_ROWS_W, 128), jnp.float32),
            pltpu.SemaphoreType.DMA,
        ],
    )
    def k(table_hbm, idx_hbm, out_hbm, idx_v, rows_v, sem):
        wid = lax.axis_index("s") * 2 + lax.axis_index("c")
        base = wid * _ROWS_W
        pltpu.sync_copy(idx_hbm.at[pl.ds(wid * _IDX_CH, _IDX_CH)], idx_v)
        copies = [
            pltpu.async_copy(
                table_hbm.at[idx_v.at[j]],
                rows_v.at[pl.ds(j * _IDX_W, _IDX_W)],
                sem,
            )
            for j in range(_IDX_CH)
        ]
        for cp in copies:
            cp.wait()
        pltpu.sync_copy(rows_v, out_hbm.at[pl.ds(base, _ROWS_W)])

    return k(table, idx2d)


def _nms_call(bcol, brow, vrow):
    return pl.pallas_call(
        _nms_body,
        out_shape=jax.ShapeDtypeStruct((1, _NP), jnp.float32),
        in_specs=[
            pl.BlockSpec(memory_space=pltpu.VMEM),
            pl.BlockSpec(memory_space=pltpu.VMEM),
            pl.BlockSpec(memory_space=pltpu.VMEM),
        ],
        out_specs=pl.BlockSpec(memory_space=pltpu.VMEM),
    )(bcol, brow, vrow)


def kernel(prediction, zone):
    pred = prediction[0]
    predt = jnp.pad(pred, ((0, _NP - _N), (0, 128 - 85))).T
    featst = _score_call(zone, predt)

    order = jnp.argsort(-featst[10, :_N], stable=True).astype(jnp.int32)
    order_p = jnp.concatenate(
        [order, jnp.full((_NP - _N,), _N, jnp.int32)])
    table = jnp.pad(featst.T, ((0, 0), (0, 112)))
    s = _gather_call(table, order_p.reshape(_NW * _IDX_CH, _IDX_W))

    bcol = jnp.stack([s[:, 1], s[:, 0], s[:, 3], s[:, 2]], axis=1)
    brow = bcol.T
    vrow = s[:, 9].reshape(1, _NP)
    keep = _nms_call(bcol, brow, vrow)

    boxes_yxyx = s[:_N, 0:4]
    in_zone = s[:_N, 4] > 0.5
    scores = s[:_N, 5]
    classes = s[:_N, 6].astype(jnp.int32)
    centers_yx = s[:_N, 7:9]
    keep_b = keep[0, :_N] > 0.5
    return (boxes_yxyx, in_zone, scores, classes, centers_yx, keep_b)
